# baseline (speedup 1.0000x reference)
import functools

import jax
import jax.numpy as jnp
from jax import lax
from jax.experimental import pallas as pl
from jax.experimental.pallas import tpu as pltpu
from jax.experimental.pallas import tpu_sc as plsc

_RS = [0.5, 1.0, 1.5, 2.0, 2.5, 3.0, 3.5, 4.0]
_ETA = [4.0, 2.0, 1.0, 0.5, 4.0, 2.0, 1.0, 0.5]
_N_SF = 8

_NC = 2
_NS = 16
_LANES = 16
_NBUF = 5


def _make_sc_fp(n_pairs: int, n_atoms: int, chunk: int):
  n_workers = _NC * _NS
  assert n_pairs % n_workers == 0
  pairs_per_tile = n_pairs // n_workers
  assert pairs_per_tile % chunk == 0
  n_chunks = pairs_per_tile // chunk
  assert n_chunks % _NBUF == 0 and n_chunks > _NBUF
  assert chunk % _LANES == 0 and chunk % 8 == 0
  n_blk = (n_atoms + 127) // 128
  n_pad = n_blk * 128
  blk_q, blk_r = divmod(n_blk, _NS)
  wbatch = 8

  mesh = plsc.VectorSubcoreMesh(core_axis_name="c", subcore_axis_name="s")

  buf_types = []
  for _ in range(_NBUF):
    buf_types += [
        pltpu.VMEM((chunk,), jnp.float32),
        pltpu.VMEM((chunk,), jnp.float32),
        pltpu.VMEM((chunk,), jnp.int32),
        pltpu.VMEM((chunk, _N_SF), jnp.float32),
        pltpu.SemaphoreType.DMA,
        pltpu.SemaphoreType.DMA,
    ]

  @functools.partial(
      pl.kernel,
      out_type=jax.ShapeDtypeStruct((_NC, n_blk, _N_SF, 128), jnp.float32),
      mesh=mesh,
      compiler_params=pltpu.CompilerParams(
          needs_layout_passes=False, use_tc_tiling_on_sc=False),
      scratch_types=[
          pltpu.VMEM_SHARED((n_pad, _N_SF), jnp.float32),
          pltpu.VMEM((wbatch * 128, _N_SF), jnp.float32),
          pltpu.VMEM((wbatch, _N_SF, 128), jnp.float32),
      ] + buf_types,
  )
  def sc_fp(dist_hbm, fc_hbm, idx_hbm, zero_hbm, part_hbm,
            acc, rowbuf, planebuf, *bufs):
    c = lax.axis_index("c")
    s = lax.axis_index("s")
    @pl.when(s == 0)
    def _():
      pltpu.sync_copy(zero_hbm, acc)
    plsc.subcore_barrier()

    wid = s * _NC + c
    base = wid * pairs_per_tile
    iota = lax.iota(jnp.int32, _LANES)
    sets = [tuple(bufs[i * 6:(i + 1) * 6]) for i in range(_NBUF)]

    def issue_in(b, k):
      dist_v, fc_v, idx_v, _, in_sem, _ = b
      off = base + k * chunk
      pltpu.async_copy(dist_hbm.at[pl.ds(off, chunk)], dist_v, in_sem)
      pltpu.async_copy(fc_hbm.at[pl.ds(off, chunk)], fc_v, in_sem)
      pltpu.async_copy(idx_hbm.at[pl.ds(off, chunk)], idx_v, in_sem)

    def wait_in(b):
      dist_v, fc_v, idx_v, _, in_sem, _ = b
      pltpu.make_async_copy(dist_hbm.at[pl.ds(0, chunk)], dist_v, in_sem).wait()
      pltpu.make_async_copy(fc_hbm.at[pl.ds(0, chunk)], fc_v, in_sem).wait()
      pltpu.make_async_copy(idx_hbm.at[pl.ds(0, chunk)], idx_v, in_sem).wait()

    def issue_stream(b):
      _, _, idx_v, sf_v, _, out_sem = b
      pltpu.async_copy(sf_v, acc.at[idx_v], out_sem, add=True)

    def wait_stream(b):
      _, _, idx_v, sf_v, _, out_sem = b
      pltpu.make_async_copy(sf_v, acc.at[idx_v], out_sem).wait()

    def compute(b):
      dist_v, fc_v, _, sf_v, _, _ = b

      @plsc.parallel_loop(0, chunk // _LANES, unroll=8)
      def grp(g):
        p0 = g * _LANES
        d = dist_v[pl.ds(p0, _LANES)]
        f = fc_v[pl.ds(p0, _LANES)]
        rows = p0 + iota
        for j in range(_N_SF):
          t = d - _RS[j]
          e = jnp.exp(t * t * (-_ETA[j])) * f
          col = jnp.full((_LANES,), j, dtype=jnp.int32)
          plsc.store_scatter(sf_v, [rows, col], e)

    issue_in(sets[0], 0)

    def outer(kk, carry):
      for p in range(_NBUF):
        k = kk * _NBUF + p
        nxt = sets[(p + 1) % _NBUF]

        @pl.when(k >= _NBUF - 1)
        def _():
          wait_stream(nxt)

        @pl.when(k + 1 < n_chunks)
        def _():
          issue_in(nxt, k + 1)

        wait_in(sets[p])
        compute(sets[p])
        issue_stream(sets[p])
      return carry

    lax.fori_loop(0, n_chunks // _NBUF, outer, 0)
    for k in range(n_chunks - (_NBUF - 1), n_chunks):
      wait_stream(sets[k % _NBUF])

    plsc.subcore_barrier()

    def wbatch_out(bb, cnt):
      pltpu.sync_copy(acc.at[pl.ds(bb * 128, cnt * 128)],
                      rowbuf.at[pl.ds(0, cnt * 128)])

      def blk(t, carry):
        t128 = t * 128
        for j in range(_N_SF):
          colj = jnp.full((_LANES,), j, dtype=jnp.int32)
          for l in range(_N_SF):
            ridx = t128 + l * _LANES + iota
            v = plsc.load_gather(rowbuf, [ridx, colj])
            planebuf[t, j, pl.ds(l * _LANES, _LANES)] = v
        return carry

      lax.fori_loop(0, cnt, blk, 0)
      pltpu.sync_copy(planebuf.at[pl.ds(0, cnt)],
                      part_hbm.at[c, pl.ds(bb, cnt)])

    def writeout(bb, cnt):
      def batch(m, carry):
        wbatch_out(bb + m * wbatch, wbatch)
        return carry

      lax.fori_loop(0, cnt // wbatch, batch, 0)
      if cnt % wbatch:
        wbatch_out(bb + (cnt // wbatch) * wbatch, cnt % wbatch)

    if blk_r:
      @pl.when(s < blk_r)
      def _():
        writeout(s * (blk_q + 1), blk_q + 1)

      @pl.when(s >= blk_r)
      def _():
        writeout(blk_r * (blk_q + 1) + (s - blk_r) * blk_q, blk_q)
    else:
      writeout(s * blk_q, blk_q)

  return sc_fp


def _reduce_body(x_ref, o_ref):
  o_ref[...] = x_ref[0] + x_ref[1]


def kernel(dist, fc, ind_2, elems):
  n_pairs = dist.shape[0]
  n_atoms = elems.shape[0]
  i_rind = ind_2[:, 0]

  chunk = 400
  n_blk = (n_atoms + 127) // 128
  zeros = jnp.zeros((n_blk * 128, _N_SF), dtype=jnp.float32)
  sc_fp = _make_sc_fp(n_pairs, n_atoms, chunk)
  partial = sc_fp(dist, fc, i_rind, zeros)

  red = pl.pallas_call(
      _reduce_body,
      out_shape=jax.ShapeDtypeStruct(partial.shape[1:], jnp.float32),
  )(partial)
  fp = jnp.transpose(red, (0, 2, 1)).reshape(n_blk * 128, _N_SF)[:n_atoms]

  p_ind = jnp.arange(n_pairs, dtype=jnp.int32)
  jacob_ind = jnp.stack([p_ind, i_rind], axis=1)
  return fp, jacob_ind


# speedup vs baseline: 2.2530x; 2.2530x over previous
import functools

import jax
import jax.numpy as jnp
from jax import lax
from jax.experimental import pallas as pl
from jax.experimental.pallas import tpu as pltpu
from jax.experimental.pallas import tpu_sc as plsc

_RS = [0.5, 1.0, 1.5, 2.0, 2.5, 3.0, 3.5, 4.0]
_ETA = [4.0, 2.0, 1.0, 0.5, 4.0, 2.0, 1.0, 0.5]
_N_SF = 8

_NC = 2
_NS = 16
_LANES = 16
_NBUF = 5


def _make_sc_fp(n_pairs: int, n_atoms: int, chunk: int):
  n_workers = _NC * _NS
  assert n_pairs % n_workers == 0
  pairs_per_tile = n_pairs // n_workers
  assert pairs_per_tile % chunk == 0
  n_chunks = pairs_per_tile // chunk
  assert n_chunks % _NBUF == 0 and n_chunks > _NBUF
  assert chunk % _LANES == 0 and chunk % 8 == 0
  n_blk = (n_atoms + 127) // 128
  n_pad = n_blk * 128
  blk_q, blk_r = divmod(n_blk, _NS)
  wbatch = 8

  mesh = plsc.VectorSubcoreMesh(core_axis_name="c", subcore_axis_name="s")

  buf_types = []
  for _ in range(_NBUF):
    buf_types += [
        pltpu.VMEM((chunk,), jnp.float32),
        pltpu.VMEM((chunk,), jnp.float32),
        pltpu.VMEM((chunk,), jnp.int32),
        pltpu.VMEM((chunk, _N_SF), jnp.float32),
        pltpu.SemaphoreType.DMA,
        pltpu.SemaphoreType.DMA,
    ]

  @functools.partial(
      pl.kernel,
      out_type=jax.ShapeDtypeStruct((_NC, n_blk, _N_SF, 128), jnp.float32),
      mesh=mesh,
      compiler_params=pltpu.CompilerParams(
          needs_layout_passes=False, use_tc_tiling_on_sc=False),
      scratch_types=[
          pltpu.VMEM_SHARED((n_pad, _N_SF), jnp.float32),
          pltpu.VMEM((wbatch * 128, _N_SF), jnp.float32),
          pltpu.VMEM((wbatch, _N_SF, 128), jnp.float32),
      ] + buf_types,
  )
  def sc_fp(dist_hbm, fc_hbm, idx_hbm, zero_hbm, part_hbm,
            acc, rowbuf, planebuf, *bufs):
    c = lax.axis_index("c")
    s = lax.axis_index("s")
    @pl.when(s == 0)
    def _():
      pltpu.sync_copy(zero_hbm, acc)
    plsc.subcore_barrier()

    wid = s * _NC + c
    base = wid * pairs_per_tile
    iota = lax.iota(jnp.int32, _LANES)
    sets = [tuple(bufs[i * 6:(i + 1) * 6]) for i in range(_NBUF)]

    def issue_in(b, k):
      dist_v, fc_v, idx_v, _, in_sem, _ = b
      off = base + k * chunk
      pltpu.async_copy(dist_hbm.at[pl.ds(off, chunk)], dist_v, in_sem)
      pltpu.async_copy(fc_hbm.at[pl.ds(off, chunk)], fc_v, in_sem)
      pltpu.async_copy(idx_hbm.at[pl.ds(off, chunk)], idx_v, in_sem)

    def wait_in(b):
      dist_v, fc_v, idx_v, _, in_sem, _ = b
      pltpu.make_async_copy(dist_hbm.at[pl.ds(0, chunk)], dist_v, in_sem).wait()
      pltpu.make_async_copy(fc_hbm.at[pl.ds(0, chunk)], fc_v, in_sem).wait()
      pltpu.make_async_copy(idx_hbm.at[pl.ds(0, chunk)], idx_v, in_sem).wait()

    def issue_stream(b):
      _, _, idx_v, sf_v, _, out_sem = b
      pltpu.async_copy(sf_v, acc.at[idx_v], out_sem, add=True)

    def wait_stream(b):
      _, _, idx_v, sf_v, _, out_sem = b
      pltpu.make_async_copy(sf_v, acc.at[idx_v], out_sem).wait()

    def compute(b):
      dist_v, fc_v, _, sf_v, _, _ = b

      @plsc.parallel_loop(0, chunk // _LANES, unroll=4)
      def grp(g):
        p0 = g * _LANES
        d = dist_v[pl.ds(p0, _LANES)]
        f = fc_v[pl.ds(p0, _LANES)]
        rows = p0 + iota
        for j in range(_N_SF):
          t = d - _RS[j]
          e = jnp.exp(t * t * (-_ETA[j])) * f
          col = jnp.full((_LANES,), j, dtype=jnp.int32)
          plsc.store_scatter(sf_v, [rows, col], e)

    issue_in(sets[0], 0)

    def outer(kk, carry):
      for p in range(_NBUF):
        k = kk * _NBUF + p
        nxt = sets[(p + 1) % _NBUF]

        @pl.when(k >= _NBUF - 1)
        def _():
          wait_stream(nxt)

        @pl.when(k + 1 < n_chunks)
        def _():
          issue_in(nxt, k + 1)

        wait_in(sets[p])
        compute(sets[p])
        issue_stream(sets[p])
      return carry

    lax.fori_loop(0, n_chunks // _NBUF, outer, 0)
    for k in range(n_chunks - (_NBUF - 1), n_chunks):
      wait_stream(sets[k % _NBUF])

    plsc.subcore_barrier()

    def wbatch_out(bb, cnt):
      pltpu.sync_copy(acc.at[pl.ds(bb * 128, cnt * 128)],
                      rowbuf.at[pl.ds(0, cnt * 128)])

      def blk(t, carry):
        t128 = t * 128
        for j in range(_N_SF):
          colj = jnp.full((_LANES,), j, dtype=jnp.int32)
          for l in range(_N_SF):
            ridx = t128 + l * _LANES + iota
            v = plsc.load_gather(rowbuf, [ridx, colj])
            planebuf[t, j, pl.ds(l * _LANES, _LANES)] = v
        return carry

      lax.fori_loop(0, cnt, blk, 0)
      pltpu.sync_copy(planebuf.at[pl.ds(0, cnt)],
                      part_hbm.at[c, pl.ds(bb, cnt)])

    def writeout(bb, cnt):
      def batch(m, carry):
        wbatch_out(bb + m * wbatch, wbatch)
        return carry

      lax.fori_loop(0, cnt // wbatch, batch, 0)
      if cnt % wbatch:
        wbatch_out(bb + (cnt // wbatch) * wbatch, cnt % wbatch)

    if blk_r:
      @pl.when(s < blk_r)
      def _():
        writeout(s * (blk_q + 1), blk_q + 1)

      @pl.when(s >= blk_r)
      def _():
        writeout(blk_r * (blk_q + 1) + (s - blk_r) * blk_q, blk_q)
    else:
      writeout(s * blk_q, blk_q)

  return sc_fp


def _reduce_body(x_ref, o_ref):
  o_ref[...] = x_ref[0] + x_ref[1]


def kernel(dist, fc, ind_2, elems):
  n_pairs = dist.shape[0]
  n_atoms = elems.shape[0]
  i_rind = ind_2[:, 0]

  chunk = 400
  n_blk = (n_atoms + 127) // 128
  zeros = jnp.zeros((n_blk * 128, _N_SF), dtype=jnp.float32)
  sc_fp = _make_sc_fp(n_pairs, n_atoms, chunk)
  partial = sc_fp(dist, fc, i_rind, zeros)

  red = pl.pallas_call(
      _reduce_body,
      out_shape=jax.ShapeDtypeStruct(partial.shape[1:], jnp.float32),
  )(partial)
  fp = jnp.transpose(red, (0, 2, 1)).reshape(n_blk * 128, _N_SF)[:n_atoms]

  p_ind = jnp.arange(n_pairs, dtype=jnp.int32)
  jacob_ind = jnp.stack([p_ind, i_rind], axis=1)
  return fp, jacob_ind
